# Initial kernel scaffold; baseline (speedup 1.0000x reference)
import jax
import jax.numpy as jnp
from jax.experimental import pallas as pl


def kernel(x, edge_val, W, edge_row, edge_col):
    raise NotImplementedError("write your pallas kernel here")


# speedup vs baseline: 3.2225x; 3.2225x over previous
import functools

import jax
import jax.numpy as jnp
from jax import lax
from jax.experimental import pallas as pl
from jax.experimental.pallas import tpu as pltpu
from jax.experimental.pallas import tpu_sc as plsc

B = 4
M = 10000
E = 320000
FIN = 128
FOUT = 128
K = 6

NC = 2
NS = 16
MP = 10240
KB = 80
SBB = 25
EPT = E // NS
NBLK = EPT // KB
NSB = NBLK // SBB
RPT = MP // NS
RP = 32
NP = RPT // RP

_f32 = jnp.float32
_i32 = jnp.int32
_GDN = lax.GatherDimensionNumbers(
    offset_dims=(), collapsed_slice_dims=(0,), start_index_map=(0,))


def _sc_body(x_hbm, ecol_hbm, erow_hbm, eval_hbm, t_hbm,
             col_s, row_s, val_s, rows_b, a_b, t1_b, t0_b, acc):
    cid = lax.axis_index("c")
    sid = lax.axis_index("s")
    row0 = sid * RPT

    def _zero_a_b():
        def zrow(r):
            for v in range(FIN // 16):
                a_b[r, pl.ds(v * 16, 16)] = jnp.zeros((16,), _f32)
        lax.fori_loop(0, RP, lambda r, c: (zrow(r), c)[1], 0)

    def _zero_acc_slice():
        _zero_a_b()
        for p in range(NP):
            pltpu.sync_copy(a_b, acc.at[pl.ds(row0 + p * RP, RP)])

    for bl in range(2):
        bb = cid * 2 + bl
        for p in range(NP):
            r0 = row0 + p * RP
            pltpu.sync_copy(x_hbm.at[bb, pl.ds(r0, RP)], t1_b)
            pltpu.sync_copy(t1_b, t_hbm.at[0, bb, pl.ds(r0, RP)])
    _zero_acc_slice()
    plsc.subcore_barrier()

    def phase(k, bl):
        b = cid * 2 + bl
        km1 = k - 1
        km2 = jnp.maximum(k - 2, 0)

        def sb_body(sb, carry):
            pltpu.sync_copy(ecol_hbm.at[sid, sb], col_s)
            pltpu.sync_copy(erow_hbm.at[sid, sb], row_s)
            pltpu.sync_copy(eval_hbm.at[sid, sb], val_s)

            def blk_body(j, c2):
                pltpu.sync_copy(t_hbm.at[km1, b].at[col_s.at[j]], rows_b)
                for g in range(KB // 16):
                    vals16 = val_s[j, pl.ds(g * 16, 16)]
                    for e16 in range(16):
                        e = g * 16 + e16
                        bval = lax.gather(
                            vals16, jnp.full((16, 1), e16, _i32),
                            _GDN, (1,),
                            mode=lax.GatherScatterMode.PROMISE_IN_BOUNDS)
                        for v in range(FIN // 16):
                            sl = pl.ds(v * 16, 16)
                            rows_b[e, sl] = rows_b[e, sl] * bval
                pltpu.sync_copy(rows_b, acc.at[row_s.at[j]], add=True)
                return c2

            return lax.fori_loop(0, SBB, blk_body, carry)

        lax.fori_loop(0, NSB, sb_body, 0)
        plsc.subcore_barrier()

        ca = jnp.where(k == 1, 1.0, 2.0).astype(_f32)
        c0 = jnp.where(k == 1, 0.0, 1.0).astype(_f32)
        for p in range(NP):
            r0 = row0 + p * RP
            pltpu.sync_copy(acc.at[pl.ds(r0, RP)], a_b)
            pltpu.sync_copy(t_hbm.at[km1, b, pl.ds(r0, RP)], t1_b)
            pltpu.sync_copy(t_hbm.at[km2, b, pl.ds(r0, RP)], t0_b)

            def drow(r):
                for v in range(FIN // 16):
                    sl = pl.ds(v * 16, 16)
                    a_b[r, sl] = (ca * a_b[r, sl] - ca * t1_b[r, sl]
                                  - c0 * t0_b[r, sl])
            lax.fori_loop(0, RP, lambda r, c: (drow(r), c)[1], 0)
            pltpu.sync_copy(a_b, t_hbm.at[k, b, pl.ds(r0, RP)])

        _zero_acc_slice()
        plsc.subcore_barrier()

    def k_body(k, carry):
        def bl_body(bl, c2):
            phase(k, bl)
            return c2
        return lax.fori_loop(0, 2, bl_body, carry)

    lax.fori_loop(1, K, k_body, 0)


_sc_cheb = functools.partial(
    pl.kernel,
    out_type=jax.ShapeDtypeStruct((K, B, MP, FIN), _f32),
    mesh=plsc.VectorSubcoreMesh(core_axis_name="c", subcore_axis_name="s"),
    scratch_types=[
        pltpu.VMEM((SBB, KB), _i32),
        pltpu.VMEM((SBB, KB), _i32),
        pltpu.VMEM((SBB, KB), _f32),
        pltpu.VMEM((KB, FIN), _f32),
        pltpu.VMEM((RP, FIN), _f32),
        pltpu.VMEM((RP, FIN), _f32),
        pltpu.VMEM((RP, FIN), _f32),
        pltpu.VMEM_SHARED((MP, FIN), _f32),
    ],
)(_sc_body)


_BM = 400


def _mm_body(t_ref, w_ref, o_ref):
    acc = jnp.zeros((_BM, FOUT), _f32)
    for k in range(K):
        acc += jnp.dot(t_ref[k, 0], w_ref[k], preferred_element_type=_f32)
    o_ref[0] = acc


def _tc_matmul(tall, wp):
    return pl.pallas_call(
        _mm_body,
        grid=(B, M // _BM),
        in_specs=[
            pl.BlockSpec((K, 1, _BM, FIN), lambda b, i: (0, b, i, 0)),
            pl.BlockSpec((K, FIN, FOUT), lambda b, i: (0, 0, 0)),
        ],
        out_specs=pl.BlockSpec((1, _BM, FOUT), lambda b, i: (b, i, 0)),
        out_shape=jax.ShapeDtypeStruct((B, M, FOUT), _f32),
    )(tall, wp)


def kernel(x, edge_val, W, edge_row, edge_col):
    xp = jnp.pad(x, ((0, 0), (0, MP - M), (0, 0)))
    ecol3 = edge_col.reshape(NS, NSB, SBB, KB)
    erow3 = edge_row.reshape(NS, NSB, SBB, KB)
    eval3 = edge_val.reshape(NS, NSB, SBB, KB)
    wp = W.reshape(FIN, K, FOUT).transpose(1, 0, 2)
    tall = _sc_cheb(xp, ecol3, erow3, eval3)
    return _tc_matmul(tall, wp)


# speedup vs baseline: 4.8617x; 1.5086x over previous
import functools

import jax
import jax.numpy as jnp
from jax import lax
from jax.experimental import pallas as pl
from jax.experimental.pallas import tpu as pltpu
from jax.experimental.pallas import tpu_sc as plsc

B = 4
M = 10000
E = 320000
FIN = 128
FOUT = 128
K = 6

NC = 2
NS = 16
MP = 10240
KB = 80
SBB = 25
EPT = E // NS
NBLK = EPT // KB
NSB = NBLK // SBB
RPT = MP // NS
RP = 32
NP = RPT // RP

_f32 = jnp.float32
_i32 = jnp.int32
_GDN = lax.GatherDimensionNumbers(
    offset_dims=(), collapsed_slice_dims=(0,), start_index_map=(0,))


def _sc_body(x_hbm, ecol_hbm, erow_hbm, eval_hbm, t_hbm,
             col_s, row_s, val_s, rows_b, a_b, t1_b, t0_b, acc, gsem):
    cid = lax.axis_index("c")
    sid = lax.axis_index("s")
    row0 = sid * RPT

    def _zero_a_b():
        def zrow(r):
            for v in range(FIN // 16):
                a_b[r, pl.ds(v * 16, 16)] = jnp.zeros((16,), _f32)
        lax.fori_loop(0, RP, lambda r, c: (zrow(r), c)[1], 0)

    def _zero_acc_slice():
        _zero_a_b()
        for p in range(NP):
            pltpu.sync_copy(a_b, acc.at[pl.ds(row0 + p * RP, RP)])

    for bl in range(2):
        bb = cid * 2 + bl
        for p in range(NP):
            r0 = row0 + p * RP
            pltpu.sync_copy(x_hbm.at[bb, pl.ds(r0, RP)], t1_b)
            pltpu.sync_copy(t1_b, t_hbm.at[0, bb, pl.ds(r0, RP)])
    _zero_acc_slice()
    plsc.subcore_barrier()

    def phase(k, bl):
        b = cid * 2 + bl
        km1 = k - 1
        km2 = jnp.maximum(k - 2, 0)

        def sb_body(sb, carry):
            pltpu.sync_copy(ecol_hbm.at[sid, sb], col_s)
            pltpu.sync_copy(erow_hbm.at[sid, sb], row_s)
            pltpu.sync_copy(eval_hbm.at[sid, sb], val_s)

            def issue(j, p):
                pltpu.async_copy(
                    t_hbm.at[km1, b].at[col_s.at[j]], rows_b.at[p], gsem)

            issue(0, 0)

            def blk_body(j, c2):
                p = lax.rem(j, 2)
                pltpu.make_async_copy(
                    t_hbm.at[km1, b].at[col_s.at[j]], rows_b.at[p],
                    gsem).wait()

                @pl.when(j < SBB - 1)
                def _():
                    issue(j + 1, 1 - p)

                for g in range(KB // 16):
                    vals16 = val_s[j, pl.ds(g * 16, 16)]
                    for e16 in range(16):
                        e = g * 16 + e16
                        bval = lax.gather(
                            vals16, jnp.full((16, 1), e16, _i32),
                            _GDN, (1,),
                            mode=lax.GatherScatterMode.PROMISE_IN_BOUNDS)
                        for v in range(FIN // 16):
                            sl = pl.ds(v * 16, 16)
                            rows_b[p, e, sl] = rows_b[p, e, sl] * bval
                pltpu.sync_copy(rows_b.at[p], acc.at[row_s.at[j]], add=True)
                return c2

            return lax.fori_loop(0, SBB, blk_body, carry)

        lax.fori_loop(0, NSB, sb_body, 0)
        plsc.subcore_barrier()

        ca = jnp.where(k == 1, 1.0, 2.0).astype(_f32)
        c0 = jnp.where(k == 1, 0.0, 1.0).astype(_f32)
        for p in range(NP):
            r0 = row0 + p * RP
            pltpu.sync_copy(acc.at[pl.ds(r0, RP)], a_b)
            pltpu.sync_copy(t_hbm.at[km1, b, pl.ds(r0, RP)], t1_b)
            pltpu.sync_copy(t_hbm.at[km2, b, pl.ds(r0, RP)], t0_b)

            def drow(r):
                for v in range(FIN // 16):
                    sl = pl.ds(v * 16, 16)
                    a_b[r, sl] = (ca * a_b[r, sl] - ca * t1_b[r, sl]
                                  - c0 * t0_b[r, sl])
            lax.fori_loop(0, RP, lambda r, c: (drow(r), c)[1], 0)
            pltpu.sync_copy(a_b, t_hbm.at[k, b, pl.ds(r0, RP)])

        _zero_acc_slice()
        plsc.subcore_barrier()

    def k_body(k, carry):
        def bl_body(bl, c2):
            phase(k, bl)
            return c2
        return lax.fori_loop(0, 2, bl_body, carry)

    lax.fori_loop(1, K, k_body, 0)


_sc_cheb = functools.partial(
    pl.kernel,
    out_type=jax.ShapeDtypeStruct((K, B, MP, FIN), _f32),
    mesh=plsc.VectorSubcoreMesh(core_axis_name="c", subcore_axis_name="s"),
    scratch_types=[
        pltpu.VMEM((SBB, KB), _i32),
        pltpu.VMEM((SBB, KB), _i32),
        pltpu.VMEM((SBB, KB), _f32),
        pltpu.VMEM((2, KB, FIN), _f32),
        pltpu.VMEM((RP, FIN), _f32),
        pltpu.VMEM((RP, FIN), _f32),
        pltpu.VMEM((RP, FIN), _f32),
        pltpu.VMEM_SHARED((MP, FIN), _f32),
        pltpu.SemaphoreType.DMA,
    ],
)(_sc_body)


_BM = 400


def _mm_body(t_ref, w_ref, o_ref):
    acc = jnp.zeros((_BM, FOUT), _f32)
    for k in range(K):
        acc += jnp.dot(t_ref[k, 0], w_ref[k], preferred_element_type=_f32)
    o_ref[0] = acc


def _tc_matmul(tall, wp):
    return pl.pallas_call(
        _mm_body,
        grid=(B, M // _BM),
        in_specs=[
            pl.BlockSpec((K, 1, _BM, FIN), lambda b, i: (0, b, i, 0)),
            pl.BlockSpec((K, FIN, FOUT), lambda b, i: (0, 0, 0)),
        ],
        out_specs=pl.BlockSpec((1, _BM, FOUT), lambda b, i: (b, i, 0)),
        out_shape=jax.ShapeDtypeStruct((B, M, FOUT), _f32),
    )(tall, wp)


def kernel(x, edge_val, W, edge_row, edge_col):
    xp = jnp.pad(x, ((0, 0), (0, MP - M), (0, 0)))
    ecol3 = edge_col.reshape(NS, NSB, SBB, KB)
    erow3 = edge_row.reshape(NS, NSB, SBB, KB)
    eval3 = edge_val.reshape(NS, NSB, SBB, KB)
    wp = W.reshape(FIN, K, FOUT).transpose(1, 0, 2)
    tall = _sc_cheb(xp, ecol3, erow3, eval3)
    return _tc_matmul(tall, wp)
